# Initial kernel scaffold; baseline (speedup 1.0000x reference)
import jax
import jax.numpy as jnp
from jax.experimental import pallas as pl


def kernel(x, targets, next_token_table):
    raise NotImplementedError("write your pallas kernel here")


# speedup vs baseline: 1.2890x; 1.2890x over previous
import functools

import jax
import jax.numpy as jnp
from jax import lax
from jax.experimental import pallas as pl
from jax.experimental.pallas import tpu as pltpu
from jax.experimental.pallas import tpu_sc as plsc

NC = 2
NS = 16
L = 16


def _lse_body(t_ref, o_ref):
    t = t_ref[...]
    m = jnp.max(t, axis=1, keepdims=True)
    s = jnp.sum(jnp.exp(t - m), axis=1, keepdims=True)
    o_ref[...] = (m + jnp.log(s))[:, 0]


def _row_lse(table):
    v, c = table.shape
    return pl.pallas_call(
        _lse_body,
        out_shape=jax.ShapeDtypeStruct((v,), jnp.float32),
    )(table)


def _make_sc_gather(n_rows, vocab, dim, dim_pad, chunk):
    n_workers = NC * NS
    per_w = n_rows // n_workers
    n_chunks = per_w // chunk
    mesh = plsc.VectorSubcoreMesh(
        core_axis_name="c", subcore_axis_name="s", num_cores=NC, num_subcores=NS
    )

    @functools.partial(
        pl.kernel,
        out_type=[
            jax.ShapeDtypeStruct((n_rows, dim), jnp.float32),
            jax.ShapeDtypeStruct((n_workers, L), jnp.float32),
        ],
        mesh=mesh,
        scratch_types=[
            pltpu.VMEM((chunk,), jnp.int32),
            pltpu.VMEM((chunk,), jnp.int32),
            pltpu.VMEM((chunk,), jnp.int32),
            pltpu.VMEM((chunk, dim), jnp.float32),
            pltpu.VMEM((chunk,), jnp.float32),
            pltpu.VMEM((chunk,), jnp.float32),
            pltpu.VMEM((L,), jnp.float32),
            pltpu.SemaphoreType.DMA,
            pltpu.SemaphoreType.DMA,
        ],
        compiler_params=pltpu.CompilerParams(use_tc_tiling_on_sc=False),
    )
    def sc_kernel(x_hbm, t_hbm, table_hbm, tabflat_hbm, lse_hbm, out_hbm,
                  part_hbm, idx_v, pick_v, tgt_v, rows_v, pval_v, lval_v,
                  acc_v, sem, sem2):
        wid = lax.axis_index("s") * NC + lax.axis_index("c")
        base0 = wid * per_w

        def body(ci, acc):
            base = base0 + ci * chunk
            pltpu.sync_copy(x_hbm.at[pl.ds(base, chunk)], idx_v)
            pltpu.sync_copy(t_hbm.at[pl.ds(base, chunk)], tgt_v)
            rows_dma = pltpu.async_copy(table_hbm.at[idx_v], rows_v, sem)
            for j in range(chunk // L):
                xs = idx_v[pl.ds(j * L, L)]
                ts = tgt_v[pl.ds(j * L, L)]
                pick_v[pl.ds(j * L, L)] = xs * dim + ts
            pltpu.async_copy(tabflat_hbm.at[pick_v], pval_v, sem2).wait()
            pltpu.async_copy(lse_hbm.at[idx_v], lval_v, sem2).wait()
            for j in range(chunk // L):
                acc = acc + lval_v[pl.ds(j * L, L)] - pval_v[pl.ds(j * L, L)]
            rows_dma.wait()
            pltpu.sync_copy(rows_v, out_hbm.at[pl.ds(base, chunk)])
            return acc

        acc = lax.fori_loop(0, n_chunks, body, jnp.zeros((L,), jnp.float32))
        acc_v[...] = acc
        pltpu.sync_copy(acc_v, part_hbm.at[wid])

    return sc_kernel


def _fin_body(p_ref, o_ref, n):
    o_ref[...] = (jnp.sum(p_ref[...]) / n).reshape(1, 1)


def kernel(x, targets, next_token_table):
    b, t = x.shape
    vocab, dim = next_token_table.shape
    n_rows = b * t

    xf = x.reshape(-1).astype(jnp.int32)
    tf = targets.reshape(-1).astype(jnp.int32)
    tabflat = jnp.pad(next_token_table.reshape(-1), (0, 8))
    lse = _row_lse(next_token_table)
    sc = _make_sc_gather(n_rows, vocab, dim, dim, chunk=32)
    logits_flat, part = sc(xf, tf, next_token_table, tabflat, lse)

    loss = pl.pallas_call(
        functools.partial(_fin_body, n=float(n_rows)),
        out_shape=jax.ShapeDtypeStruct((1, 1), jnp.float32),
    )(part)

    return logits_flat.reshape(b, t, dim), loss[0, 0]


# speedup vs baseline: 2.5013x; 1.9405x over previous
import functools

import jax
import jax.numpy as jnp
from jax import lax
from jax.experimental import pallas as pl
from jax.experimental.pallas import tpu as pltpu
from jax.experimental.pallas import tpu_sc as plsc

NC = 2
NS = 16
L = 16
SPLIT = 896


def _lse_body(t_ref, o_ref):
    t = t_ref[...]
    m = jnp.max(t, axis=1, keepdims=True)
    s = jnp.sum(jnp.exp(t - m), axis=1, keepdims=True)
    o_ref[...] = (m + jnp.log(s))[:, 0]


def _row_lse(table):
    v, c = table.shape
    return pl.pallas_call(
        _lse_body,
        out_shape=jax.ShapeDtypeStruct((v,), jnp.float32),
    )(table)


def _make_mesh():
    return plsc.VectorSubcoreMesh(
        core_axis_name="c", subcore_axis_name="s", num_cores=NC, num_subcores=NS
    )


def _make_sc_loss(n_rows, vocab, dim):
    n_workers = NC * NS
    per_w = n_rows // n_workers

    @functools.partial(
        pl.kernel,
        out_type=jax.ShapeDtypeStruct((n_workers, L), jnp.float32),
        mesh=_make_mesh(),
        scratch_types=[
            pltpu.VMEM((per_w,), jnp.int32),
            pltpu.VMEM((per_w,), jnp.int32),
            pltpu.VMEM((per_w,), jnp.int32),
            pltpu.VMEM((per_w,), jnp.float32),
            pltpu.VMEM((per_w,), jnp.float32),
            pltpu.VMEM((L,), jnp.float32),
            pltpu.SemaphoreType.DMA,
        ],
        compiler_params=pltpu.CompilerParams(use_tc_tiling_on_sc=False),
    )
    def loss_kernel(x_hbm, t_hbm, tabflat_hbm, lse_hbm, part_hbm,
                    idx_v, tgt_v, pick_v, pval_v, lval_v, acc_v, sem):
        wid = lax.axis_index("s") * NC + lax.axis_index("c")
        base0 = wid * per_w
        pltpu.sync_copy(x_hbm.at[pl.ds(base0, per_w)], idx_v)
        pltpu.sync_copy(t_hbm.at[pl.ds(base0, per_w)], tgt_v)

        def pick_body(i, _):
            o = i * L
            pick_v[pl.ds(o, L)] = idx_v[pl.ds(o, L)] * dim + tgt_v[pl.ds(o, L)]
            return 0

        lax.fori_loop(0, per_w // L, pick_body, 0)
        pltpu.async_copy(tabflat_hbm.at[pick_v], pval_v, sem).wait()
        pltpu.async_copy(lse_hbm.at[idx_v], lval_v, sem).wait()

        def acc_body(i, acc):
            o = i * L
            return acc + lval_v[pl.ds(o, L)] - pval_v[pl.ds(o, L)]

        acc = lax.fori_loop(0, per_w // L, acc_body,
                            jnp.zeros((L,), jnp.float32))
        acc_v[...] = acc
        pltpu.sync_copy(acc_v, part_hbm.at[wid])

    return loss_kernel


def _make_sc_gather(b, t_pad, vocab, dim):
    n_workers = NC * NS
    b_per_w = b // n_workers
    dim_pad = SPLIT + 128

    @functools.partial(
        pl.kernel,
        out_type=jax.ShapeDtypeStruct((b, t_pad, dim_pad), jnp.float32),
        mesh=_make_mesh(),
        scratch_types=[
            pltpu.VMEM((t_pad,), jnp.int32),
            pltpu.VMEM((t_pad,), jnp.int32),
            pltpu.VMEM((t_pad, dim_pad), jnp.float32),
            pltpu.VMEM((t_pad, dim_pad), jnp.float32),
            pltpu.SemaphoreType.DMA,
            pltpu.SemaphoreType.DMA,
        ],
    )
    def gather_kernel(x2_hbm, ta_hbm, tb_hbm, out_hbm,
                      idx0_v, idx1_v, rows0_v, rows1_v, sem0, sem1):
        wid = lax.axis_index("s") * NC + lax.axis_index("c")
        bbase = wid * b_per_w
        idxs = (idx0_v, idx1_v)
        bufs = (rows0_v, rows1_v)
        sems = (sem0, sem1)

        def start(k):
            p = k % 2
            pltpu.sync_copy(x2_hbm.at[bbase + k], idxs[p])
            da = pltpu.async_copy(
                ta_hbm.at[idxs[p]], bufs[p].at[:, pl.ds(0, SPLIT)],
                sems[p])
            db = pltpu.async_copy(
                tb_hbm.at[idxs[p]], bufs[p].at[:, pl.ds(SPLIT, 128)],
                sems[p])
            return (da, db)

        dmas = {0: start(0)}
        for k in range(b_per_w):
            p = k % 2
            if k + 1 < b_per_w:
                dmas[k + 1] = start(k + 1)
            da, db = dmas.pop(k)
            da.wait()
            db.wait()
            pltpu.sync_copy(bufs[p], out_hbm.at[bbase + k])

    return gather_kernel


def _fin_body(p_ref, o_ref, n):
    o_ref[...] = (jnp.sum(p_ref[...]) / n).reshape(1, 1)


def kernel(x, targets, next_token_table):
    b, t = x.shape
    vocab, dim = next_token_table.shape
    n_rows = b * t

    xf = x.reshape(-1).astype(jnp.int32)
    tf = targets.reshape(-1).astype(jnp.int32)
    tabflat = jnp.pad(next_token_table.reshape(-1), (0, 8))
    tab_a = next_token_table[:, :SPLIT]
    tab_b = jnp.pad(next_token_table[:, SPLIT:],
                    ((0, 0), (0, 128 - (dim - SPLIT))))

    lse = _row_lse(next_token_table)
    part = _make_sc_loss(n_rows, vocab, dim)(xf, tf, tabflat, lse)
    t_pad = -(-t // 8) * 8
    xp = jnp.concatenate(
        [x.astype(jnp.int32), x[:, 2 * t - t_pad:].astype(jnp.int32)], axis=1)
    out_pad = _make_sc_gather(b, t_pad, vocab, dim)(xp, tab_a, tab_b)
    logits = out_pad[:, :t, :dim]

    loss = pl.pallas_call(
        functools.partial(_fin_body, n=float(n_rows)),
        out_shape=jax.ShapeDtypeStruct((1, 1), jnp.float32),
    )(part)

    return logits, loss[0, 0]
